# baseline (speedup 1.0000x reference)
import functools

import jax
import jax.numpy as jnp
from jax import lax
from jax.experimental import pallas as pl
from jax.experimental.pallas import tpu as pltpu
from jax.experimental.pallas import tpu_sc as plsc

D = 128
ROW_TILE = 800

CHUNK = 80
RING = 3
NB = 12
CPB = 21
ROUNDS = CPB // RING
CPT = NB * CPB
N_TILES = 16
N_PAD = 10112
RPT = N_PAD // N_TILES


def _dyn_bcast(vals16, e16):
    idx = jnp.full((16, 1), e16, jnp.int32)
    return lax.gather(
        vals16, idx,
        dimension_numbers=lax.GatherDimensionNumbers(
            offset_dims=(), collapsed_slice_dims=(0,), start_index_map=(0,)),
        slice_sizes=(1,),
        mode=lax.GatherScatterMode.PROMISE_IN_BOUNDS)


def _mm_body(x_ref, b_ref, w_ref, o_ref, bi_ref):
    o_ref[:] = jnp.dot(x_ref[:], w_ref[:], preferred_element_type=jnp.float32)

    @pl.when(pl.program_id(0) == 0)
    def _():
        bi_ref[:] = jnp.broadcast_to(b_ref[:], (RPT, D))


def _support(x2d, w, bias):
    m = x2d.shape[0]
    return pl.pallas_call(
        _mm_body,
        grid=(m // ROW_TILE,),
        in_specs=[
            pl.BlockSpec((ROW_TILE, D), lambda i: (i, 0)),
            pl.BlockSpec((1, D), lambda i: (0, 0)),
            pl.BlockSpec((D, D), lambda i: (0, 0)),
        ],
        out_specs=[
            pl.BlockSpec((ROW_TILE, D), lambda i: (i, 0)),
            pl.BlockSpec((RPT, D), lambda i: (0, 0)),
        ],
        out_shape=[
            jax.ShapeDtypeStruct((m, D), jnp.float32),
            jax.ShapeDtypeStruct((RPT, D), jnp.float32),
        ],
    )(x2d, bias.reshape(1, D), w)


def _spmm(sup, row1, col1, val1, biasimg, n_nodes, n_batch):
    ept = row1.shape[0] // N_TILES
    epb = CPB * CHUNK
    mesh = plsc.VectorSubcoreMesh(core_axis_name="c", subcore_axis_name="s")

    @functools.partial(
        pl.kernel,
        out_type=jax.ShapeDtypeStruct((n_batch * N_PAD, D), jnp.float32),
        mesh=mesh,
        scratch_types=[
            [pltpu.VMEM((epb,), jnp.int32) for _ in range(2)],
            [pltpu.VMEM((epb,), jnp.int32) for _ in range(2)],
            [pltpu.VMEM((epb,), jnp.float32) for _ in range(2)],
            [pltpu.VMEM((CHUNK,), jnp.int32) for _ in range(RING)],
            [pltpu.VMEM((CHUNK,), jnp.int32) for _ in range(RING)],
            [pltpu.VMEM((CHUNK, D), jnp.float32) for _ in range(RING)],
            [pltpu.SemaphoreType.DMA for _ in range(RING)],
            [pltpu.SemaphoreType.DMA for _ in range(RING)],
            [pltpu.SemaphoreType.DMA for _ in range(3)],
            pltpu.VMEM_SHARED((N_PAD, D), jnp.float32),
        ],
    )
    def k(sup_hbm, row_hbm, col_hbm, val_hbm, bi_hbm, out_hbm,
          row_blk, col_blk, val_blk, rowc, colc, rows, gsem, ssem, isem,
          acc):
        c = lax.axis_index("c")
        s = lax.axis_index("s")

        def g_issue(sl):
            pltpu.async_copy(sup_hbm.at[colc[sl]], rows[sl], gsem[sl])

        def g_wait(sl):
            pltpu.make_async_copy(
                sup_hbm.at[colc[sl]], rows[sl], gsem[sl]).wait()

        def w_issue(sl):
            pltpu.async_copy(rows[sl], acc.at[rowc[sl]], ssem[sl], add=True)

        def w_wait(sl):
            pltpu.make_async_copy(
                rows[sl], acc.at[rowc[sl]], ssem[sl]).wait()

        def idx_issue(base, pk):
            pltpu.async_copy(row_hbm.at[pl.ds(base, epb)], row_blk[pk],
                             isem[0])
            pltpu.async_copy(col_hbm.at[pl.ds(base, epb)], col_blk[pk],
                             isem[1])
            pltpu.async_copy(val_hbm.at[pl.ds(base, epb)], val_blk[pk],
                             isem[2])

        def idx_wait(base, pk):
            pltpu.make_async_copy(row_hbm.at[pl.ds(base, epb)], row_blk[pk],
                                  isem[0]).wait()
            pltpu.make_async_copy(col_hbm.at[pl.ds(base, epb)], col_blk[pk],
                                  isem[1]).wait()
            pltpu.make_async_copy(val_hbm.at[pl.ds(base, epb)], val_blk[pk],
                                  isem[2]).wait()

        for p in range(2):
            b = c * 2 + p
            b_n = b * n_nodes

            def stage(jj, sl, pk):
                eo = jj * CHUNK
                for f in range(CHUNK // 16):
                    colc[sl][pl.ds(f * 16, 16)] = (
                        col_blk[pk][pl.ds(eo + f * 16, 16)] + b_n)
                    rowc[sl][pl.ds(f * 16, 16)] = (
                        row_blk[pk][pl.ds(eo + f * 16, 16)])

            def scale(jj, sl, pk):
                def grp(g, carry):
                    vals16 = val_blk[pk][pl.ds(jj * CHUNK + g * 16, 16)]

                    def edge(e16, carry2):
                        vv = _dyn_bcast(vals16, e16)
                        e = g * 16 + e16
                        for f in range(D // 16):
                            rows[sl][e, pl.ds(f * 16, 16)] = (
                                rows[sl][e, pl.ds(f * 16, 16)] * vv)
                        return carry2

                    lax.fori_loop(0, 16, edge, None)
                    return carry

                lax.fori_loop(0, CHUNK // 16, grp, None)

            def slot_step(jj, sl, pk, do_wwait, stage_next):
                g_wait(sl)
                scale(jj, sl, pk)
                w_issue(sl)
                prev = (sl + 2) % RING
                if do_wwait:
                    w_wait(prev)
                if stage_next is not None:
                    njj, npk = stage_next
                    stage(njj, prev, npk)
                    g_issue(prev)

            pltpu.sync_copy(bi_hbm, acc.at[pl.ds(s * RPT, RPT)])
            plsc.subcore_barrier()

            base0 = s * ept
            idx_issue(base0, 0)
            idx_wait(base0, 0)
            stage(0, 0, 0)
            g_issue(0)
            stage(1, 1, 0)
            g_issue(1)

            def rnd(r, carry, pk):
                j0 = r * RING
                slot_step(j0, 0, pk,
                          do_wwait=True, stage_next=(j0 + 2, pk))
                slot_step(j0 + 1, 1, pk,
                          do_wwait=True, stage_next=(j0 + 3, pk))
                slot_step(j0 + 2, 2, pk,
                          do_wwait=True, stage_next=(j0 + 4, pk))
                return carry

            def tail_round(pk, npk, nbase, last):
                j0 = (ROUNDS - 1) * RING
                slot_step(j0, 0, pk, do_wwait=True,
                          stage_next=(j0 + 2, pk))
                if not last:
                    idx_wait(nbase, npk)
                    slot_step(j0 + 1, 1, pk, do_wwait=True,
                              stage_next=(0, npk))
                    slot_step(j0 + 2, 2, pk, do_wwait=True,
                              stage_next=(1, npk))
                else:
                    slot_step(j0 + 1, 1, pk, do_wwait=True, stage_next=None)
                    slot_step(j0 + 2, 2, pk, do_wwait=True, stage_next=None)

            def mid_block(kb, pk):
                nbase = base0 + (kb + 1) * epb
                idx_issue(nbase, 1 - pk)
                lax.fori_loop(0, ROUNDS - 1,
                              functools.partial(rnd, pk=pk), None)
                tail_round(pk, 1 - pk, nbase, last=False)

            idx_issue(base0 + epb, 1)
            slot_step(0, 0, 0, do_wwait=False, stage_next=(2, 0))
            slot_step(1, 1, 0, do_wwait=True, stage_next=(3, 0))
            slot_step(2, 2, 0, do_wwait=True, stage_next=(4, 0))
            lax.fori_loop(1, ROUNDS - 1, functools.partial(rnd, pk=0), None)
            tail_round(0, 1, base0 + epb, last=False)

            def blkpair(q, carry):
                mid_block(1 + 2 * q, 1)
                mid_block(2 + 2 * q, 0)
                return carry

            lax.fori_loop(0, (NB - 2) // 2, blkpair, None)

            lax.fori_loop(0, ROUNDS - 1, functools.partial(rnd, pk=1), None)
            tail_round(1, 0, 0, last=True)

            w_wait(2)
            plsc.subcore_barrier()
            pltpu.sync_copy(
                acc.at[pl.ds(s * RPT, RPT)],
                out_hbm.at[pl.ds(b * N_PAD + s * RPT, RPT)])
            plsc.subcore_barrier()

    return k(sup, row1, col1, val1, biasimg)


def kernel(adj_row, adj_col, adj_val, input_feature, weight, bias):
    n_batch, n_nodes, d_in = input_feature.shape
    sup, biasimg = _support(
        input_feature.reshape(n_batch * n_nodes, d_in), weight, bias)
    n_edges = adj_row.shape[0]
    e_pad = N_TILES * CPT * CHUNK - n_edges
    row1 = jnp.concatenate(
        [adj_row.astype(jnp.int32), jnp.zeros((e_pad,), jnp.int32)])
    col1 = jnp.concatenate(
        [adj_col.astype(jnp.int32), jnp.zeros((e_pad,), jnp.int32)])
    val1 = jnp.concatenate([adj_val, jnp.zeros((e_pad,), jnp.float32)])
    out = _spmm(sup, row1, col1, val1, biasimg, n_nodes, n_batch)
    out = out.reshape(n_batch, N_PAD, D)[:, :n_nodes, :]
    return out


# speedup vs baseline: 1.0218x; 1.0218x over previous
import functools

import jax
import jax.numpy as jnp
from jax import lax
from jax.experimental import pallas as pl
from jax.experimental.pallas import tpu as pltpu
from jax.experimental.pallas import tpu_sc as plsc

D = 128
ROW_TILE = 800

CHUNK = 112
RING = 3
NB = 30
CPB = 6
ROUNDS = CPB // RING
CPT = NB * CPB
N_TILES = 16
N_PAD = 10112
RPT = N_PAD // N_TILES


def _dyn_bcast(vals16, e16):
    idx = jnp.full((16, 1), e16, jnp.int32)
    return lax.gather(
        vals16, idx,
        dimension_numbers=lax.GatherDimensionNumbers(
            offset_dims=(), collapsed_slice_dims=(0,), start_index_map=(0,)),
        slice_sizes=(1,),
        mode=lax.GatherScatterMode.PROMISE_IN_BOUNDS)


def _mm_body(x_ref, b_ref, w_ref, o_ref, bi_ref):
    o_ref[:] = jnp.dot(x_ref[:], w_ref[:], preferred_element_type=jnp.float32)

    @pl.when(pl.program_id(0) == 0)
    def _():
        bi_ref[:] = jnp.broadcast_to(b_ref[:], (RPT, D))


def _support(x2d, w, bias):
    m = x2d.shape[0]
    return pl.pallas_call(
        _mm_body,
        grid=(m // ROW_TILE,),
        in_specs=[
            pl.BlockSpec((ROW_TILE, D), lambda i: (i, 0)),
            pl.BlockSpec((1, D), lambda i: (0, 0)),
            pl.BlockSpec((D, D), lambda i: (0, 0)),
        ],
        out_specs=[
            pl.BlockSpec((ROW_TILE, D), lambda i: (i, 0)),
            pl.BlockSpec((RPT, D), lambda i: (0, 0)),
        ],
        out_shape=[
            jax.ShapeDtypeStruct((m, D), jnp.float32),
            jax.ShapeDtypeStruct((RPT, D), jnp.float32),
        ],
    )(x2d, bias.reshape(1, D), w)


def _spmm(sup, row1, col1, val1, biasimg, n_nodes, n_batch):
    ept = row1.shape[0] // N_TILES
    epb = CPB * CHUNK
    mesh = plsc.VectorSubcoreMesh(core_axis_name="c", subcore_axis_name="s")

    @functools.partial(
        pl.kernel,
        out_type=jax.ShapeDtypeStruct((n_batch * N_PAD, D), jnp.float32),
        mesh=mesh,
        scratch_types=[
            [pltpu.VMEM((epb,), jnp.int32) for _ in range(2)],
            [pltpu.VMEM((epb,), jnp.int32) for _ in range(2)],
            [pltpu.VMEM((epb,), jnp.float32) for _ in range(2)],
            [pltpu.VMEM((CHUNK,), jnp.int32) for _ in range(RING)],
            [pltpu.VMEM((CHUNK,), jnp.int32) for _ in range(RING)],
            [pltpu.VMEM((CHUNK, D), jnp.float32) for _ in range(RING)],
            [pltpu.SemaphoreType.DMA for _ in range(RING)],
            [pltpu.SemaphoreType.DMA for _ in range(RING)],
            [pltpu.SemaphoreType.DMA for _ in range(3)],
            pltpu.VMEM_SHARED((N_PAD, D), jnp.float32),
        ],
    )
    def k(sup_hbm, row_hbm, col_hbm, val_hbm, bi_hbm, out_hbm,
          row_blk, col_blk, val_blk, rowc, colc, rows, gsem, ssem, isem,
          acc):
        c = lax.axis_index("c")
        s = lax.axis_index("s")

        def g_issue(sl):
            pltpu.async_copy(sup_hbm.at[colc[sl]], rows[sl], gsem[sl])

        def g_wait(sl):
            pltpu.make_async_copy(
                sup_hbm.at[colc[sl]], rows[sl], gsem[sl]).wait()

        def w_issue(sl):
            pltpu.async_copy(rows[sl], acc.at[rowc[sl]], ssem[sl], add=True)

        def w_wait(sl):
            pltpu.make_async_copy(
                rows[sl], acc.at[rowc[sl]], ssem[sl]).wait()

        def idx_issue(base, pk):
            pltpu.async_copy(row_hbm.at[pl.ds(base, epb)], row_blk[pk],
                             isem[0])
            pltpu.async_copy(col_hbm.at[pl.ds(base, epb)], col_blk[pk],
                             isem[1])
            pltpu.async_copy(val_hbm.at[pl.ds(base, epb)], val_blk[pk],
                             isem[2])

        def idx_wait(base, pk):
            pltpu.make_async_copy(row_hbm.at[pl.ds(base, epb)], row_blk[pk],
                                  isem[0]).wait()
            pltpu.make_async_copy(col_hbm.at[pl.ds(base, epb)], col_blk[pk],
                                  isem[1]).wait()
            pltpu.make_async_copy(val_hbm.at[pl.ds(base, epb)], val_blk[pk],
                                  isem[2]).wait()

        for p in range(2):
            b = c * 2 + p
            b_n = b * n_nodes

            def stage(jj, sl, pk):
                eo = jj * CHUNK
                for f in range(CHUNK // 16):
                    colc[sl][pl.ds(f * 16, 16)] = (
                        col_blk[pk][pl.ds(eo + f * 16, 16)] + b_n)
                    rowc[sl][pl.ds(f * 16, 16)] = (
                        row_blk[pk][pl.ds(eo + f * 16, 16)])

            def scale(jj, sl, pk):
                def grp(g, carry):
                    vals16 = val_blk[pk][pl.ds(jj * CHUNK + g * 16, 16)]

                    def edge(e16, carry2):
                        vv = _dyn_bcast(vals16, e16)
                        e = g * 16 + e16
                        for f in range(D // 16):
                            rows[sl][e, pl.ds(f * 16, 16)] = (
                                rows[sl][e, pl.ds(f * 16, 16)] * vv)
                        return carry2

                    lax.fori_loop(0, 16, edge, None)
                    return carry

                lax.fori_loop(0, CHUNK // 16, grp, None)

            def slot_step(jj, sl, pk, do_wwait, stage_next):
                g_wait(sl)
                scale(jj, sl, pk)
                w_issue(sl)
                prev = (sl + 2) % RING
                if do_wwait:
                    w_wait(prev)
                if stage_next is not None:
                    njj, npk = stage_next
                    stage(njj, prev, npk)
                    g_issue(prev)

            pltpu.sync_copy(bi_hbm, acc.at[pl.ds(s * RPT, RPT)])
            plsc.subcore_barrier()

            base0 = s * ept
            idx_issue(base0, 0)
            idx_wait(base0, 0)
            stage(0, 0, 0)
            g_issue(0)
            stage(1, 1, 0)
            g_issue(1)

            def rnd(r, carry, pk):
                j0 = r * RING
                slot_step(j0, 0, pk,
                          do_wwait=True, stage_next=(j0 + 2, pk))
                slot_step(j0 + 1, 1, pk,
                          do_wwait=True, stage_next=(j0 + 3, pk))
                slot_step(j0 + 2, 2, pk,
                          do_wwait=True, stage_next=(j0 + 4, pk))
                return carry

            def tail_round(pk, npk, nbase, last):
                j0 = (ROUNDS - 1) * RING
                slot_step(j0, 0, pk, do_wwait=True,
                          stage_next=(j0 + 2, pk))
                if not last:
                    idx_wait(nbase, npk)
                    slot_step(j0 + 1, 1, pk, do_wwait=True,
                              stage_next=(0, npk))
                    slot_step(j0 + 2, 2, pk, do_wwait=True,
                              stage_next=(1, npk))
                else:
                    slot_step(j0 + 1, 1, pk, do_wwait=True, stage_next=None)
                    slot_step(j0 + 2, 2, pk, do_wwait=True, stage_next=None)

            def mid_block(kb, pk):
                nbase = base0 + (kb + 1) * epb
                idx_issue(nbase, 1 - pk)
                lax.fori_loop(0, ROUNDS - 1,
                              functools.partial(rnd, pk=pk), None)
                tail_round(pk, 1 - pk, nbase, last=False)

            idx_issue(base0 + epb, 1)
            slot_step(0, 0, 0, do_wwait=False, stage_next=(2, 0))
            slot_step(1, 1, 0, do_wwait=True, stage_next=(3, 0))
            slot_step(2, 2, 0, do_wwait=True, stage_next=(4, 0))
            lax.fori_loop(1, ROUNDS - 1, functools.partial(rnd, pk=0), None)
            tail_round(0, 1, base0 + epb, last=False)

            def blkpair(q, carry):
                mid_block(1 + 2 * q, 1)
                mid_block(2 + 2 * q, 0)
                return carry

            lax.fori_loop(0, (NB - 2) // 2, blkpair, None)

            lax.fori_loop(0, ROUNDS - 1, functools.partial(rnd, pk=1), None)
            tail_round(1, 0, 0, last=True)

            w_wait(2)
            plsc.subcore_barrier()
            pltpu.sync_copy(
                acc.at[pl.ds(s * RPT, RPT)],
                out_hbm.at[pl.ds(b * N_PAD + s * RPT, RPT)])
            plsc.subcore_barrier()

    return k(sup, row1, col1, val1, biasimg)


def kernel(adj_row, adj_col, adj_val, input_feature, weight, bias):
    n_batch, n_nodes, d_in = input_feature.shape
    sup, biasimg = _support(
        input_feature.reshape(n_batch * n_nodes, d_in), weight, bias)
    n_edges = adj_row.shape[0]
    e_pad = N_TILES * CPT * CHUNK - n_edges
    row1 = jnp.concatenate(
        [adj_row.astype(jnp.int32), jnp.zeros((e_pad,), jnp.int32)])
    col1 = jnp.concatenate(
        [adj_col.astype(jnp.int32), jnp.zeros((e_pad,), jnp.int32)])
    val1 = jnp.concatenate([adj_val, jnp.zeros((e_pad,), jnp.float32)])
    out = _spmm(sup, row1, col1, val1, biasimg, n_nodes, n_batch)
    out = out.reshape(n_batch, N_PAD, D)[:, :n_nodes, :]
    return out
